# baseline (speedup 1.0000x reference)
import functools

import jax
import jax.numpy as jnp
from jax import lax
from jax.experimental import pallas as pl
from jax.experimental.pallas import tpu as pltpu
from jax.experimental.pallas import tpu_sc as plsc

N = 10000
E = 320000
D = 128
NPAD = 10240
NC, NS, L = 2, 16, 16
NW = NC * NS
EPW = E // NW
KCH, BCH = 125, 80
RPT = NPAD // NS

_mesh = plsc.VectorSubcoreMesh(core_axis_name="c", subcore_axis_name="s",
                               num_cores=NC, num_subcores=NS)
_sc_params = pltpu.CompilerParams(needs_layout_passes=False)


@functools.partial(
    pl.kernel,
    out_type=jax.ShapeDtypeStruct((NW, NPAD), jnp.float32),
    mesh=_mesh,
    compiler_params=_sc_params,
    scratch_types=[
        pltpu.VMEM((EPW,), jnp.int32),
        pltpu.VMEM((NPAD,), jnp.float32),
    ],
)
def _sc_degree(dst_hbm, deg_hbm, idx_v, hist_v):
    c = lax.axis_index("c")
    s = lax.axis_index("s")
    wid = c * NS + s
    zero16 = jnp.zeros((L,), jnp.float32)
    one16 = jnp.ones((L,), jnp.float32)

    def _zero(i, carry):
        hist_v[pl.ds(i * L, L)] = zero16
        return carry

    lax.fori_loop(0, NPAD // L, _zero, 0)
    pltpu.sync_copy(dst_hbm.at[pl.ds(wid * EPW, EPW)], idx_v)

    def _hist(i, carry):
        idx = idx_v[pl.ds(i * L, L)]
        plsc.addupdate_scatter(hist_v, [idx], one16)
        return carry

    lax.fori_loop(0, EPW // L, _hist, 0)
    pltpu.sync_copy(hist_v, deg_hbm.at[wid])


def _tc_transform_body(x_ref, w_ref, deg_ref, g_ref):
    deg = jnp.sum(deg_ref[...], axis=0) + 1.0
    inv = lax.rsqrt(jnp.maximum(deg, 1e-9))
    h = jnp.dot(x_ref[...], w_ref[...], preferred_element_type=jnp.float32)
    g_ref[...] = h * inv[:, None]


def _tc_transform(x2p, W, deg_parts):
    blk = 1024
    grid = NPAD // blk
    return pl.pallas_call(
        _tc_transform_body,
        grid=(grid,),
        in_specs=[
            pl.BlockSpec((blk, D), lambda i: (i, 0)),
            pl.BlockSpec((D, D), lambda i: (0, 0)),
            pl.BlockSpec((NW, blk), lambda i: (0, i)),
        ],
        out_specs=pl.BlockSpec((blk, D), lambda i: (i, 0)),
        out_shape=jax.ShapeDtypeStruct((NPAD, D), jnp.float32),
    )(x2p, W, deg_parts)


@functools.partial(
    pl.kernel,
    out_type=jax.ShapeDtypeStruct((NC, NPAD, D), jnp.float32),
    mesh=_mesh,
    compiler_params=_sc_params,
    scratch_types=[
        pltpu.VMEM((EPW,), jnp.int32),
        pltpu.VMEM((KCH, BCH), jnp.int32),
        pltpu.VMEM((BCH, D), jnp.float32),
        pltpu.VMEM((BCH, D), jnp.float32),
        pltpu.SemaphoreType.DMA,
        pltpu.SemaphoreType.DMA,
        pltpu.SemaphoreType.DMA,
        pltpu.SemaphoreType.DMA,
        pltpu.VMEM_SHARED((NPAD, D), jnp.float32),
    ],
)
def _sc_scatter(g_hbm, src_hbm, dst_hbm, part_hbm, isrc_v, idst_v, buf0_v,
                buf1_v, semg0, semg1, sems0, sems1, acc_sh):
    c = lax.axis_index("c")
    s = lax.axis_index("s")
    wid = c * NS + s
    rbase = s * RPT

    pltpu.sync_copy(src_hbm.at[pl.ds(wid * EPW, EPW)], isrc_v)
    pltpu.sync_copy(dst_hbm.at[wid], idst_v)

    pltpu.sync_copy(g_hbm.at[pl.ds(rbase, RPT)],
                    acc_sh.at[pl.ds(rbase, RPT)])

    plsc.subcore_barrier()

    def _gather(j, buf, semg):
        pltpu.async_copy(g_hbm.at[isrc_v.at[pl.ds(j * BCH, BCH)]], buf, semg)

    def _gwait(j, buf, semg):
        pltpu.make_async_copy(g_hbm.at[isrc_v.at[pl.ds(j * BCH, BCH)]],
                              buf, semg).wait()

    def _scat(j, buf, sems):
        pltpu.async_copy(buf, acc_sh.at[idst_v.at[j]], sems, add=True)

    def _swait(j, buf, sems):
        pltpu.make_async_copy(buf, acc_sh.at[idst_v.at[j]], sems).wait()

    _gather(0, buf0_v, semg0)
    _gather(1, buf1_v, semg1)

    def _edge(t, carry):
        a = 2 * t
        _gwait(a, buf0_v, semg0)
        _scat(a, buf0_v, sems0)
        _gwait(a + 1, buf1_v, semg1)
        _scat(a + 1, buf1_v, sems1)

        @pl.when(a + 2 < KCH)
        def _():
            _swait(a, buf0_v, sems0)
            _gather(a + 2, buf0_v, semg0)

        @pl.when(a + 3 < KCH)
        def _():
            _swait(a + 1, buf1_v, sems1)
            _gather(a + 3, buf1_v, semg1)

        return carry

    lax.fori_loop(0, KCH // 2, _edge, 0)

    _gwait(KCH - 1, buf0_v, semg0)
    _scat(KCH - 1, buf0_v, sems0)
    _swait(KCH - 2, buf1_v, sems1)
    _swait(KCH - 1, buf0_v, sems0)

    plsc.subcore_barrier()
    pltpu.sync_copy(acc_sh.at[pl.ds(rbase, RPT)],
                    part_hbm.at[c, pl.ds(rbase, RPT)])


def _tc_finish_body(part_ref, g_ref, deg_ref, b_ref, out_ref):
    deg = jnp.sum(deg_ref[...], axis=0) + 1.0
    inv = lax.rsqrt(jnp.maximum(deg, 1e-9))
    p = part_ref[...]
    out_ref[...] = (p[0] + p[1] - g_ref[...]) * inv[:, None] + b_ref[...]


def _tc_finish(part, g, deg_parts, b2):
    blk = 1024
    grid = NPAD // blk
    return pl.pallas_call(
        _tc_finish_body,
        grid=(grid,),
        in_specs=[
            pl.BlockSpec((NC, blk, D), lambda i: (0, i, 0)),
            pl.BlockSpec((blk, D), lambda i: (i, 0)),
            pl.BlockSpec((NW, blk), lambda i: (0, i)),
            pl.BlockSpec((1, D), lambda i: (0, 0)),
        ],
        out_specs=pl.BlockSpec((blk, D), lambda i: (i, 0)),
        out_shape=jax.ShapeDtypeStruct((N, D), jnp.float32),
    )(part, g, deg_parts, b2)


def kernel(x, edge_index, W, b):
    x2 = x.reshape(N, D)
    src_flat = edge_index[0]
    dst3 = edge_index[1].reshape(NW, KCH, BCH)
    dst_flat = edge_index[1]
    b2 = b.reshape(1, D)

    deg_parts = _sc_degree(dst_flat)
    g = _tc_transform(x2, W, deg_parts)
    part = _sc_scatter(g, src_flat, dst3)
    outp = _tc_finish(part, g, deg_parts, b2)
    return outp.reshape(1, 1, N, D)


# speedup vs baseline: 1.1907x; 1.1907x over previous
import functools

import jax
import jax.numpy as jnp
from jax import lax
from jax.experimental import pallas as pl
from jax.experimental.pallas import tpu as pltpu
from jax.experimental.pallas import tpu_sc as plsc

N = 10000
E = 320000
D = 128
NPAD = 10240
NC, NS, L = 2, 16, 16
NW = NC * NS
EPW = E // NW
KCH, BCH = 125, 80
RPT = NPAD // NS

_mesh = plsc.VectorSubcoreMesh(core_axis_name="c", subcore_axis_name="s",
                               num_cores=NC, num_subcores=NS)
_sc_params = pltpu.CompilerParams(needs_layout_passes=False)


@functools.partial(
    pl.kernel,
    out_type=jax.ShapeDtypeStruct((NW, NPAD), jnp.float32),
    mesh=_mesh,
    compiler_params=_sc_params,
    scratch_types=[
        pltpu.VMEM((EPW,), jnp.int32),
        pltpu.VMEM((NPAD,), jnp.float32),
    ],
)
def _sc_degree(dst_hbm, deg_hbm, idx_v, hist_v):
    c = lax.axis_index("c")
    s = lax.axis_index("s")
    wid = c * NS + s
    zero16 = jnp.zeros((L,), jnp.float32)
    one16 = jnp.ones((L,), jnp.float32)

    def _zero(i, carry):
        hist_v[pl.ds(i * L, L)] = zero16
        return carry

    lax.fori_loop(0, NPAD // L, _zero, 0)
    pltpu.sync_copy(dst_hbm.at[pl.ds(wid * EPW, EPW)], idx_v)

    def _hist(i, carry):
        idx = idx_v[pl.ds(i * L, L)]
        plsc.addupdate_scatter(hist_v, [idx], one16)
        return carry

    lax.fori_loop(0, EPW // L, _hist, 0)
    pltpu.sync_copy(hist_v, deg_hbm.at[wid])


def _tc_transform_body(x_ref, w_ref, deg_ref, g_ref):
    deg = jnp.sum(deg_ref[...], axis=0) + 1.0
    inv = lax.rsqrt(jnp.maximum(deg, 1e-9))
    h = jnp.dot(x_ref[...], w_ref[...], preferred_element_type=jnp.float32)
    g_ref[...] = h * inv[:, None]


def _tc_transform(x2p, W, deg_parts):
    blk = 1024
    grid = NPAD // blk
    return pl.pallas_call(
        _tc_transform_body,
        grid=(grid,),
        in_specs=[
            pl.BlockSpec((blk, D), lambda i: (i, 0)),
            pl.BlockSpec((D, D), lambda i: (0, 0)),
            pl.BlockSpec((NW, blk), lambda i: (0, i)),
        ],
        out_specs=pl.BlockSpec((blk, D), lambda i: (i, 0)),
        out_shape=jax.ShapeDtypeStruct((NPAD, D), jnp.float32),
    )(x2p, W, deg_parts)


@functools.partial(
    pl.kernel,
    out_type=jax.ShapeDtypeStruct((NC, NPAD, D), jnp.float32),
    mesh=_mesh,
    compiler_params=_sc_params,
    scratch_types=[
        pltpu.VMEM((EPW,), jnp.int32),
        pltpu.VMEM((KCH, BCH), jnp.int32),
        pltpu.VMEM((BCH, D), jnp.float32),
        pltpu.VMEM((BCH, D), jnp.float32),
        pltpu.SemaphoreType.DMA,
        pltpu.SemaphoreType.DMA,
        pltpu.SemaphoreType.DMA,
        pltpu.SemaphoreType.DMA,
        pltpu.VMEM_SHARED((NPAD, D), jnp.float32),
    ],
)
def _sc_scatter(g_hbm, src_hbm, dst_hbm, part_hbm, isrc_v, idst_v, buf0_v,
                buf1_v, semg0, semg1, sems0, sems1, acc_sh):
    c = lax.axis_index("c")
    s = lax.axis_index("s")
    wid = c * NS + s
    rbase = s * RPT

    pltpu.sync_copy(src_hbm.at[pl.ds(wid * EPW, EPW)], isrc_v)
    pltpu.sync_copy(dst_hbm.at[wid], idst_v)

    pltpu.sync_copy(g_hbm.at[pl.ds(rbase, RPT)],
                    acc_sh.at[pl.ds(rbase, RPT)])

    plsc.subcore_barrier()

    def _gather(j, buf, semg):
        pltpu.async_copy(g_hbm.at[isrc_v.at[pl.ds(j * BCH, BCH)]], buf, semg)

    def _gwait(j, buf, semg):
        pltpu.make_async_copy(g_hbm.at[isrc_v.at[pl.ds(j * BCH, BCH)]],
                              buf, semg).wait()

    def _scat(j, buf, sems):
        pltpu.async_copy(buf, acc_sh.at[idst_v.at[j]], sems, add=True)

    def _swait(j, buf, sems):
        pltpu.make_async_copy(buf, acc_sh.at[idst_v.at[j]], sems).wait()

    _gather(0, buf0_v, semg0)

    def _edge(t, carry):
        a = 2 * t
        _gather(a + 1, buf1_v, semg1)
        _gwait(a, buf0_v, semg0)
        pltpu.sync_copy(buf0_v, acc_sh.at[idst_v.at[a]], add=True)

        @pl.when(a + 2 < KCH)
        def _():
            _gather(a + 2, buf0_v, semg0)

        _gwait(a + 1, buf1_v, semg1)
        pltpu.sync_copy(buf1_v, acc_sh.at[idst_v.at[a + 1]], add=True)
        return carry

    lax.fori_loop(0, KCH // 2, _edge, 0)

    _gwait(KCH - 1, buf0_v, semg0)
    pltpu.sync_copy(buf0_v, acc_sh.at[idst_v.at[KCH - 1]], add=True)

    plsc.subcore_barrier()
    pltpu.sync_copy(acc_sh.at[pl.ds(rbase, RPT)],
                    part_hbm.at[c, pl.ds(rbase, RPT)])


def _tc_finish_body(part_ref, g_ref, deg_ref, b_ref, out_ref):
    deg = jnp.sum(deg_ref[...], axis=0) + 1.0
    inv = lax.rsqrt(jnp.maximum(deg, 1e-9))
    p = part_ref[...]
    out_ref[...] = (p[0] + p[1] - g_ref[...]) * inv[:, None] + b_ref[...]


def _tc_finish(part, g, deg_parts, b2):
    blk = 1024
    grid = NPAD // blk
    return pl.pallas_call(
        _tc_finish_body,
        grid=(grid,),
        in_specs=[
            pl.BlockSpec((NC, blk, D), lambda i: (0, i, 0)),
            pl.BlockSpec((blk, D), lambda i: (i, 0)),
            pl.BlockSpec((NW, blk), lambda i: (0, i)),
            pl.BlockSpec((1, D), lambda i: (0, 0)),
        ],
        out_specs=pl.BlockSpec((blk, D), lambda i: (i, 0)),
        out_shape=jax.ShapeDtypeStruct((N, D), jnp.float32),
    )(part, g, deg_parts, b2)


def kernel(x, edge_index, W, b):
    x2 = x.reshape(N, D)
    src_flat = edge_index[0]
    dst3 = edge_index[1].reshape(NW, KCH, BCH)
    dst_flat = edge_index[1]
    b2 = b.reshape(1, D)

    deg_parts = _sc_degree(dst_flat)
    g = _tc_transform(x2, W, deg_parts)
    part = _sc_scatter(g, src_flat, dst3)
    outp = _tc_finish(part, g, deg_parts, b2)
    return outp.reshape(1, 1, N, D)


# speedup vs baseline: 1.2027x; 1.0101x over previous
import functools

import jax
import jax.numpy as jnp
from jax import lax
from jax.experimental import pallas as pl
from jax.experimental.pallas import tpu as pltpu
from jax.experimental.pallas import tpu_sc as plsc

N = 10000
E = 320000
D = 128
NPAD = 10240
NC, NS, L = 2, 16, 16
NW = NC * NS
EPW = E // NW
KCH, BCH = 125, 80
RPT = NPAD // NS

_mesh = plsc.VectorSubcoreMesh(core_axis_name="c", subcore_axis_name="s",
                               num_cores=NC, num_subcores=NS)
_sc_params = pltpu.CompilerParams(needs_layout_passes=False)


@functools.partial(
    pl.kernel,
    out_type=jax.ShapeDtypeStruct((NW, NPAD), jnp.float32),
    mesh=_mesh,
    compiler_params=_sc_params,
    scratch_types=[
        pltpu.VMEM((EPW,), jnp.int32),
        pltpu.VMEM((NPAD,), jnp.float32),
    ],
)
def _sc_degree(dst_hbm, deg_hbm, idx_v, hist_v):
    c = lax.axis_index("c")
    s = lax.axis_index("s")
    wid = c * NS + s
    zero16 = jnp.zeros((L,), jnp.float32)
    one16 = jnp.ones((L,), jnp.float32)

    def _zero(i, carry):
        hist_v[pl.ds(i * L, L)] = zero16
        return carry

    lax.fori_loop(0, NPAD // L, _zero, 0)
    pltpu.sync_copy(dst_hbm.at[pl.ds(wid * EPW, EPW)], idx_v)

    def _hist(i, carry):
        idx = idx_v[pl.ds(i * L, L)]
        plsc.addupdate_scatter(hist_v, [idx], one16)
        return carry

    lax.fori_loop(0, EPW // L, _hist, 0)
    pltpu.sync_copy(hist_v, deg_hbm.at[wid])


def _tc_transform_body(x_ref, w_ref, deg_ref, g_ref):
    deg = jnp.sum(deg_ref[...], axis=0) + 1.0
    inv = lax.rsqrt(jnp.maximum(deg, 1e-9))
    h = jnp.dot(x_ref[...], w_ref[...], preferred_element_type=jnp.float32)
    g_ref[...] = h * inv[:, None]


def _tc_transform(x2p, W, deg_parts):
    blk = 1024
    grid = NPAD // blk
    return pl.pallas_call(
        _tc_transform_body,
        grid=(grid,),
        in_specs=[
            pl.BlockSpec((blk, D), lambda i: (i, 0)),
            pl.BlockSpec((D, D), lambda i: (0, 0)),
            pl.BlockSpec((NW, blk), lambda i: (0, i)),
        ],
        out_specs=pl.BlockSpec((blk, D), lambda i: (i, 0)),
        out_shape=jax.ShapeDtypeStruct((NPAD, D), jnp.float32),
    )(x2p, W, deg_parts)


@functools.partial(
    pl.kernel,
    out_type=jax.ShapeDtypeStruct((NC, NPAD, D), jnp.float32),
    mesh=_mesh,
    compiler_params=_sc_params,
    scratch_types=[
        pltpu.VMEM((EPW,), jnp.int32),
        pltpu.VMEM((KCH, BCH), jnp.int32),
        pltpu.VMEM((BCH, D), jnp.float32),
        pltpu.VMEM((BCH, D), jnp.float32),
        pltpu.SemaphoreType.DMA,
        pltpu.SemaphoreType.DMA,
        pltpu.VMEM_SHARED((NPAD, D), jnp.float32),
    ],
)
def _sc_scatter(g_hbm, src_hbm, dst3_hbm, part_hbm, isrc_v, idst_v, buf0_v,
                buf1_v, semg0, semg1, acc_sh):
    c = lax.axis_index("c")
    s = lax.axis_index("s")
    wid = c * NS + s
    rbase = s * RPT

    pltpu.sync_copy(src_hbm.at[pl.ds(wid * EPW, EPW)], isrc_v)
    pltpu.sync_copy(dst3_hbm.at[wid], idst_v)

    pltpu.async_copy(g_hbm.at[isrc_v.at[pl.ds(0, BCH)]], buf0_v, semg0)
    pltpu.async_copy(g_hbm.at[isrc_v.at[pl.ds(BCH, BCH)]], buf1_v, semg1)

    pltpu.sync_copy(g_hbm.at[pl.ds(rbase, RPT)],
                    acc_sh.at[pl.ds(rbase, RPT)])

    plsc.subcore_barrier()

    def _gather(j, buf, semg):
        pltpu.async_copy(g_hbm.at[isrc_v.at[pl.ds(j * BCH, BCH)]], buf, semg)

    def _gwait(j, buf, semg):
        pltpu.make_async_copy(g_hbm.at[isrc_v.at[pl.ds(j * BCH, BCH)]],
                              buf, semg).wait()

    def _edge(t, carry):
        a = 2 * t
        _gwait(a, buf0_v, semg0)
        pltpu.sync_copy(buf0_v, acc_sh.at[idst_v.at[a]], add=True)

        @pl.when(a + 2 < KCH)
        def _():
            _gather(a + 2, buf0_v, semg0)

        _gwait(a + 1, buf1_v, semg1)
        pltpu.sync_copy(buf1_v, acc_sh.at[idst_v.at[a + 1]], add=True)

        @pl.when(a + 3 < KCH)
        def _():
            _gather(a + 3, buf1_v, semg1)

        return carry

    lax.fori_loop(0, KCH // 2, _edge, 0)

    _gwait(KCH - 1, buf0_v, semg0)
    pltpu.sync_copy(buf0_v, acc_sh.at[idst_v.at[KCH - 1]], add=True)

    plsc.subcore_barrier()
    pltpu.sync_copy(acc_sh.at[pl.ds(rbase, RPT)],
                    part_hbm.at[c, pl.ds(rbase, RPT)])


def _tc_finish_body(part_ref, g_ref, deg_ref, b_ref, out_ref):
    deg = jnp.sum(deg_ref[...], axis=0) + 1.0
    inv = lax.rsqrt(jnp.maximum(deg, 1e-9))
    p = part_ref[...]
    out_ref[...] = (p[0] + p[1] - g_ref[...]) * inv[:, None] + b_ref[...]


def _tc_finish(part, g, deg_parts, b2):
    blk = 1024
    grid = NPAD // blk
    return pl.pallas_call(
        _tc_finish_body,
        grid=(grid,),
        in_specs=[
            pl.BlockSpec((NC, blk, D), lambda i: (0, i, 0)),
            pl.BlockSpec((blk, D), lambda i: (i, 0)),
            pl.BlockSpec((NW, blk), lambda i: (0, i)),
            pl.BlockSpec((1, D), lambda i: (0, 0)),
        ],
        out_specs=pl.BlockSpec((blk, D), lambda i: (i, 0)),
        out_shape=jax.ShapeDtypeStruct((N, D), jnp.float32),
    )(part, g, deg_parts, b2)


def kernel(x, edge_index, W, b):
    x2 = x.reshape(N, D)
    src_flat = edge_index[0]
    dst_flat = edge_index[1]
    dst3 = dst_flat.reshape(NW, KCH, BCH)
    b2 = b.reshape(1, D)

    deg_parts = _sc_degree(dst_flat)
    g = _tc_transform(x2, W, deg_parts)
    part = _sc_scatter(g, src_flat, dst3)
    outp = _tc_finish(part, g, deg_parts, b2)
    return outp.reshape(1, 1, N, D)


# speedup vs baseline: 1.2546x; 1.0431x over previous
import functools

import jax
import jax.numpy as jnp
from jax import lax
from jax.experimental import pallas as pl
from jax.experimental.pallas import tpu as pltpu
from jax.experimental.pallas import tpu_sc as plsc

N = 10000
E = 320000
D = 128
NPAD = 10240
NC, NS, L = 2, 16, 16
NW = NC * NS
EPW = E // NW
KCH, BCH = 125, 80
RPT = NPAD // NS
EPA = 9984
ERE = E - NW * EPA

_mesh = plsc.VectorSubcoreMesh(core_axis_name="c", subcore_axis_name="s",
                               num_cores=NC, num_subcores=NS)
_sc_params = pltpu.CompilerParams(needs_layout_passes=False)


@functools.partial(
    pl.kernel,
    out_type=jax.ShapeDtypeStruct((NW, NPAD), jnp.float32),
    mesh=_mesh,
    compiler_params=_sc_params,
    scratch_types=[
        pltpu.VMEM((2, EPA), jnp.int32),
        pltpu.VMEM((2, ERE), jnp.int32),
        pltpu.VMEM((NPAD,), jnp.float32),
    ],
)
def _sc_degree(ei_hbm, deg_hbm, slab_v, slab2_v, hist_v):
    c = lax.axis_index("c")
    s = lax.axis_index("s")
    wid = c * NS + s
    zero16 = jnp.zeros((L,), jnp.float32)
    one16 = jnp.ones((L,), jnp.float32)

    def _zero(i, carry):
        hist_v[pl.ds(i * L, L)] = zero16
        return carry

    lax.fori_loop(0, NPAD // L, _zero, 0)
    pltpu.sync_copy(ei_hbm.at[:, pl.ds(wid * EPA, EPA)], slab_v)

    def _hist(i, carry):
        idx = slab_v[1, pl.ds(i * L, L)]
        plsc.addupdate_scatter(hist_v, [idx], one16)
        return carry

    lax.fori_loop(0, EPA // L, _hist, 0)

    @pl.when(wid == NW - 1)
    def _():
        pltpu.sync_copy(ei_hbm.at[:, pl.ds(NW * EPA, ERE)], slab2_v)

        def _hist2(i, carry):
            idx = slab2_v[1, pl.ds(i * L, L)]
            plsc.addupdate_scatter(hist_v, [idx], one16)
            return carry

        lax.fori_loop(0, ERE // L, _hist2, 0)

    pltpu.sync_copy(hist_v, deg_hbm.at[wid])


def _tc_transform_body(x_ref, w_ref, deg_ref, g_ref):
    deg = jnp.sum(deg_ref[...], axis=0) + 1.0
    inv = lax.rsqrt(jnp.maximum(deg, 1e-9))
    h = jnp.dot(x_ref[...], w_ref[...], preferred_element_type=jnp.float32)
    g_ref[...] = h * inv[:, None]


def _tc_transform(x2p, W, deg_parts):
    blk = 1024
    grid = NPAD // blk
    return pl.pallas_call(
        _tc_transform_body,
        grid=(grid,),
        in_specs=[
            pl.BlockSpec((blk, D), lambda i: (i, 0)),
            pl.BlockSpec((D, D), lambda i: (0, 0)),
            pl.BlockSpec((NW, blk), lambda i: (0, i)),
        ],
        out_specs=pl.BlockSpec((blk, D), lambda i: (i, 0)),
        out_shape=jax.ShapeDtypeStruct((NPAD, D), jnp.float32),
    )(x2p, W, deg_parts)


@functools.partial(
    pl.kernel,
    out_type=jax.ShapeDtypeStruct((NC, NPAD, D), jnp.float32),
    mesh=_mesh,
    compiler_params=_sc_params,
    scratch_types=[
        pltpu.VMEM((EPW,), jnp.int32),
        pltpu.VMEM((KCH, BCH), jnp.int32),
        pltpu.VMEM((BCH, D), jnp.float32),
        pltpu.VMEM((BCH, D), jnp.float32),
        pltpu.SemaphoreType.DMA,
        pltpu.SemaphoreType.DMA,
        pltpu.VMEM_SHARED((NPAD, D), jnp.float32),
    ],
)
def _sc_scatter(g_hbm, src_hbm, dst3_hbm, part_hbm, isrc_v, idst_v, buf0_v,
                buf1_v, semg0, semg1, acc_sh):
    c = lax.axis_index("c")
    s = lax.axis_index("s")
    wid = c * NS + s
    rbase = s * RPT

    pltpu.sync_copy(src_hbm.at[pl.ds(wid * EPW, EPW)], isrc_v)
    pltpu.sync_copy(dst3_hbm.at[wid], idst_v)

    pltpu.async_copy(g_hbm.at[isrc_v.at[pl.ds(0, BCH)]], buf0_v, semg0)
    pltpu.async_copy(g_hbm.at[isrc_v.at[pl.ds(BCH, BCH)]], buf1_v, semg1)

    pltpu.sync_copy(g_hbm.at[pl.ds(rbase, RPT)],
                    acc_sh.at[pl.ds(rbase, RPT)])

    plsc.subcore_barrier()

    def _gather(j, buf, semg):
        pltpu.async_copy(g_hbm.at[isrc_v.at[pl.ds(j * BCH, BCH)]], buf, semg)

    def _gwait(j, buf, semg):
        pltpu.make_async_copy(g_hbm.at[isrc_v.at[pl.ds(j * BCH, BCH)]],
                              buf, semg).wait()

    def _edge(t, carry):
        a = 2 * t
        _gwait(a, buf0_v, semg0)
        pltpu.sync_copy(buf0_v, acc_sh.at[idst_v.at[a]], add=True)

        @pl.when(a + 2 < KCH)
        def _():
            _gather(a + 2, buf0_v, semg0)

        _gwait(a + 1, buf1_v, semg1)
        pltpu.sync_copy(buf1_v, acc_sh.at[idst_v.at[a + 1]], add=True)

        @pl.when(a + 3 < KCH)
        def _():
            _gather(a + 3, buf1_v, semg1)

        return carry

    lax.fori_loop(0, KCH // 2, _edge, 0)

    _gwait(KCH - 1, buf0_v, semg0)
    pltpu.sync_copy(buf0_v, acc_sh.at[idst_v.at[KCH - 1]], add=True)

    plsc.subcore_barrier()
    pltpu.sync_copy(acc_sh.at[pl.ds(rbase, RPT)],
                    part_hbm.at[c, pl.ds(rbase, RPT)])


def _tc_finish_body(part_ref, g_ref, deg_ref, b_ref, out_ref):
    deg = jnp.sum(deg_ref[...], axis=0) + 1.0
    inv = lax.rsqrt(jnp.maximum(deg, 1e-9))
    p = part_ref[...]
    out_ref[...] = (p[0] + p[1] - g_ref[...]) * inv[:, None] + b_ref[...]


def _tc_finish(part, g, deg_parts, b2):
    blk = 1024
    grid = NPAD // blk
    return pl.pallas_call(
        _tc_finish_body,
        grid=(grid,),
        in_specs=[
            pl.BlockSpec((NC, blk, D), lambda i: (0, i, 0)),
            pl.BlockSpec((blk, D), lambda i: (i, 0)),
            pl.BlockSpec((NW, blk), lambda i: (0, i)),
            pl.BlockSpec((1, D), lambda i: (0, 0)),
        ],
        out_specs=pl.BlockSpec((blk, D), lambda i: (i, 0)),
        out_shape=jax.ShapeDtypeStruct((N, D), jnp.float32),
    )(part, g, deg_parts, b2)


def kernel(x, edge_index, W, b):
    x2 = x.reshape(N, D)
    src_flat = edge_index[0]
    dst_flat = edge_index[1]
    dst3 = dst_flat.reshape(NW, KCH, BCH)
    b2 = b.reshape(1, D)

    deg_parts = _sc_degree(edge_index)
    g = _tc_transform(x2, W, deg_parts)
    part = _sc_scatter(g, src_flat, dst3)
    outp = _tc_finish(part, g, deg_parts, b2)
    return outp.reshape(1, 1, N, D)


# speedup vs baseline: 1.2584x; 1.0031x over previous
import functools

import jax
import jax.numpy as jnp
from jax import lax
from jax.experimental import pallas as pl
from jax.experimental.pallas import tpu as pltpu
from jax.experimental.pallas import tpu_sc as plsc

N = 10000
E = 320000
D = 128
NPAD = 10240
NC, NS, L = 2, 16, 16
NW = NC * NS
EPW = E // NW
KCH, BCH = 125, 80
RPT = NPAD // NS
EPA = 9984
ERE = E - NW * EPA

_mesh = plsc.VectorSubcoreMesh(core_axis_name="c", subcore_axis_name="s",
                               num_cores=NC, num_subcores=NS)
_sc_params = pltpu.CompilerParams(needs_layout_passes=False)


@functools.partial(
    pl.kernel,
    out_type=jax.ShapeDtypeStruct((NW, NPAD), jnp.float32),
    mesh=_mesh,
    compiler_params=_sc_params,
    scratch_types=[
        pltpu.VMEM((2, EPA), jnp.int32),
        pltpu.VMEM((2, ERE), jnp.int32),
        pltpu.VMEM((NPAD,), jnp.float32),
    ],
)
def _sc_degree(ei_hbm, deg_hbm, slab_v, slab2_v, hist_v):
    c = lax.axis_index("c")
    s = lax.axis_index("s")
    wid = c * NS + s
    zero16 = jnp.zeros((L,), jnp.float32)
    one16 = jnp.ones((L,), jnp.float32)

    def _zero(i, carry):
        hist_v[pl.ds(i * L, L)] = zero16
        return carry

    lax.fori_loop(0, NPAD // L, _zero, 0)
    pltpu.sync_copy(ei_hbm.at[:, pl.ds(wid * EPA, EPA)], slab_v)

    def _hist(i, carry):
        idx = slab_v[1, pl.ds(i * L, L)]
        plsc.addupdate_scatter(hist_v, [idx], one16)
        return carry

    lax.fori_loop(0, EPA // L, _hist, 0)

    @pl.when(wid == NW - 1)
    def _():
        pltpu.sync_copy(ei_hbm.at[:, pl.ds(NW * EPA, ERE)], slab2_v)

        def _hist2(i, carry):
            idx = slab2_v[1, pl.ds(i * L, L)]
            plsc.addupdate_scatter(hist_v, [idx], one16)
            return carry

        lax.fori_loop(0, ERE // L, _hist2, 0)

    pltpu.sync_copy(hist_v, deg_hbm.at[wid])


def _tc_transform_body(x_ref, w_ref, deg_ref, g_ref):
    deg = jnp.sum(deg_ref[...], axis=0) + 1.0
    inv = lax.rsqrt(jnp.maximum(deg, 1e-9))
    h = jnp.dot(x_ref[...], w_ref[...], preferred_element_type=jnp.float32)
    g_ref[...] = h * inv[:, None]


def _tc_transform(x2p, W, deg_parts):
    blk = 1024
    grid = NPAD // blk
    return pl.pallas_call(
        _tc_transform_body,
        grid=(grid,),
        in_specs=[
            pl.BlockSpec((blk, D), lambda i: (i, 0)),
            pl.BlockSpec((D, D), lambda i: (0, 0)),
            pl.BlockSpec((NW, blk), lambda i: (0, i)),
        ],
        out_specs=pl.BlockSpec((blk, D), lambda i: (i, 0)),
        out_shape=jax.ShapeDtypeStruct((NPAD, D), jnp.float32),
    )(x2p, W, deg_parts)


@functools.partial(
    pl.kernel,
    out_type=jax.ShapeDtypeStruct((NC, NPAD, D), jnp.float32),
    mesh=_mesh,
    compiler_params=_sc_params,
    scratch_types=[
        pltpu.VMEM((EPW,), jnp.int32),
        pltpu.VMEM((KCH, BCH), jnp.int32),
        pltpu.VMEM((BCH, D), jnp.float32),
        pltpu.VMEM((BCH, D), jnp.float32),
        pltpu.SemaphoreType.DMA,
        pltpu.SemaphoreType.DMA,
        pltpu.VMEM_SHARED((NPAD, D), jnp.float32),
    ],
)
def _sc_scatter(g_hbm, src_hbm, dst3_hbm, part_hbm, isrc_v, idst_v, buf0_v,
                buf1_v, semg0, semg1, acc_sh):
    c = lax.axis_index("c")
    s = lax.axis_index("s")
    wid = c * NS + s
    rbase = s * RPT

    pltpu.sync_copy(src_hbm.at[pl.ds(wid * EPW, EPW)], isrc_v)
    pltpu.sync_copy(dst3_hbm.at[wid], idst_v)

    @pl.when(c == 0)
    def _():
        pltpu.async_copy(g_hbm.at[isrc_v.at[pl.ds(0, BCH)]], buf0_v, semg0)
        pltpu.async_copy(g_hbm.at[isrc_v.at[pl.ds(BCH, BCH)]], buf1_v, semg1)
        pltpu.sync_copy(g_hbm.at[pl.ds(rbase, RPT)],
                        acc_sh.at[pl.ds(rbase, RPT)])

    @pl.when(c != 0)
    def _():
        zero16 = jnp.zeros((L,), jnp.float32)

        def _z(i, carry):
            buf0_v[lax.div(i, jnp.int32(D // L)),
                   pl.ds(lax.rem(i, jnp.int32(D // L)) * L, L)] = zero16
            return carry

        lax.fori_loop(0, BCH * D // L, _z, 0)

        def _zcp(k, carry):
            pltpu.sync_copy(buf0_v, acc_sh.at[pl.ds(rbase + k * BCH, BCH)])
            return carry

        lax.fori_loop(0, RPT // BCH, _zcp, 0)
        pltpu.async_copy(g_hbm.at[isrc_v.at[pl.ds(0, BCH)]], buf0_v, semg0)
        pltpu.async_copy(g_hbm.at[isrc_v.at[pl.ds(BCH, BCH)]], buf1_v, semg1)

    plsc.subcore_barrier()

    def _gather(j, buf, semg):
        pltpu.async_copy(g_hbm.at[isrc_v.at[pl.ds(j * BCH, BCH)]], buf, semg)

    def _gwait(j, buf, semg):
        pltpu.make_async_copy(g_hbm.at[isrc_v.at[pl.ds(j * BCH, BCH)]],
                              buf, semg).wait()

    def _edge(t, carry):
        a = 2 * t
        _gwait(a, buf0_v, semg0)
        pltpu.sync_copy(buf0_v, acc_sh.at[idst_v.at[a]], add=True)

        @pl.when(a + 2 < KCH)
        def _():
            _gather(a + 2, buf0_v, semg0)

        _gwait(a + 1, buf1_v, semg1)
        pltpu.sync_copy(buf1_v, acc_sh.at[idst_v.at[a + 1]], add=True)

        @pl.when(a + 3 < KCH)
        def _():
            _gather(a + 3, buf1_v, semg1)

        return carry

    lax.fori_loop(0, KCH // 2, _edge, 0)

    _gwait(KCH - 1, buf0_v, semg0)
    pltpu.sync_copy(buf0_v, acc_sh.at[idst_v.at[KCH - 1]], add=True)

    plsc.subcore_barrier()
    pltpu.sync_copy(acc_sh.at[pl.ds(rbase, RPT)],
                    part_hbm.at[c, pl.ds(rbase, RPT)])


def _tc_finish_body(part_ref, deg_ref, b_ref, out_ref):
    deg = jnp.sum(deg_ref[...], axis=0) + 1.0
    inv = lax.rsqrt(jnp.maximum(deg, 1e-9))
    p = part_ref[...]
    out_ref[...] = (p[0] + p[1]) * inv[:, None] + b_ref[...]


def _tc_finish(part, deg_parts, b2):
    blk = 1024
    grid = NPAD // blk
    return pl.pallas_call(
        _tc_finish_body,
        grid=(grid,),
        in_specs=[
            pl.BlockSpec((NC, blk, D), lambda i: (0, i, 0)),
            pl.BlockSpec((NW, blk), lambda i: (0, i)),
            pl.BlockSpec((1, D), lambda i: (0, 0)),
        ],
        out_specs=pl.BlockSpec((blk, D), lambda i: (i, 0)),
        out_shape=jax.ShapeDtypeStruct((N, D), jnp.float32),
    )(part, deg_parts, b2)


def kernel(x, edge_index, W, b):
    x2 = x.reshape(N, D)
    src_flat = edge_index[0]
    dst_flat = edge_index[1]
    dst3 = dst_flat.reshape(NW, KCH, BCH)
    b2 = b.reshape(1, D)

    deg_parts = _sc_degree(edge_index)
    g = _tc_transform(x2, W, deg_parts)
    part = _sc_scatter(g, src_flat, dst3)
    outp = _tc_finish(part, deg_parts, b2)
    return outp.reshape(1, 1, N, D)


# speedup vs baseline: 1.2926x; 1.0271x over previous
import functools

import jax
import jax.numpy as jnp
from jax import lax
from jax.experimental import pallas as pl
from jax.experimental.pallas import tpu as pltpu
from jax.experimental.pallas import tpu_sc as plsc

N = 10000
E = 320000
D = 128
NPAD = 10240
NC, NS, L = 2, 16, 16
NW = NC * NS
EPW = E // NW
KCH, BCH = 125, 80
RPT = NPAD // NS
EPA = 9984
ERE = E - NW * EPA

_mesh = plsc.VectorSubcoreMesh(core_axis_name="c", subcore_axis_name="s",
                               num_cores=NC, num_subcores=NS)
_sc_params = pltpu.CompilerParams(needs_layout_passes=False)


@functools.partial(
    pl.kernel,
    out_type=jax.ShapeDtypeStruct((NW, NPAD), jnp.float32),
    mesh=_mesh,
    compiler_params=_sc_params,
    scratch_types=[
        pltpu.VMEM((2, EPA), jnp.int32),
        pltpu.VMEM((2, ERE), jnp.int32),
        pltpu.VMEM((NPAD,), jnp.float32),
    ],
)
def _sc_degree(ei_hbm, deg_hbm, slab_v, slab2_v, hist_v):
    c = lax.axis_index("c")
    s = lax.axis_index("s")
    wid = c * NS + s
    zero16 = jnp.zeros((L,), jnp.float32)
    one16 = jnp.ones((L,), jnp.float32)

    def _zero(i, carry):
        hist_v[pl.ds(i * L, L)] = zero16
        return carry

    lax.fori_loop(0, NPAD // L, _zero, 0)
    pltpu.sync_copy(ei_hbm.at[:, pl.ds(wid * EPA, EPA)], slab_v)

    def _hist(i, carry):
        idx = slab_v[1, pl.ds(i * L, L)]
        plsc.addupdate_scatter(hist_v, [idx], one16)
        return carry

    lax.fori_loop(0, EPA // L, _hist, 0)

    @pl.when(wid == NW - 1)
    def _():
        pltpu.sync_copy(ei_hbm.at[:, pl.ds(NW * EPA, ERE)], slab2_v)

        def _hist2(i, carry):
            idx = slab2_v[1, pl.ds(i * L, L)]
            plsc.addupdate_scatter(hist_v, [idx], one16)
            return carry

        lax.fori_loop(0, ERE // L, _hist2, 0)

    pltpu.sync_copy(hist_v, deg_hbm.at[wid])


def _tc_transform_body(x_ref, w_ref, deg_ref, g_ref):
    deg = jnp.sum(deg_ref[...], axis=0) + 1.0
    inv = lax.rsqrt(jnp.maximum(deg, 1e-9))
    h = jnp.dot(x_ref[...], w_ref[...], preferred_element_type=jnp.float32)
    g_ref[...] = h * inv[:, None]


def _tc_transform(x2p, W, deg_parts):
    blk = 2048
    grid = NPAD // blk
    return pl.pallas_call(
        _tc_transform_body,
        grid=(grid,),
        in_specs=[
            pl.BlockSpec((blk, D), lambda i: (i, 0)),
            pl.BlockSpec((D, D), lambda i: (0, 0)),
            pl.BlockSpec((NW, blk), lambda i: (0, i)),
        ],
        out_specs=pl.BlockSpec((blk, D), lambda i: (i, 0)),
        out_shape=jax.ShapeDtypeStruct((NPAD, D), jnp.float32),
    )(x2p, W, deg_parts)


@functools.partial(
    pl.kernel,
    out_type=jax.ShapeDtypeStruct((NC, NPAD, D), jnp.float32),
    mesh=_mesh,
    compiler_params=_sc_params,
    scratch_types=[
        pltpu.VMEM((EPW,), jnp.int32),
        pltpu.VMEM((KCH, BCH), jnp.int32),
        pltpu.VMEM((BCH, D), jnp.float32),
        pltpu.VMEM((BCH, D), jnp.float32),
        pltpu.SemaphoreType.DMA,
        pltpu.SemaphoreType.DMA,
        pltpu.VMEM_SHARED((NPAD, D), jnp.float32),
    ],
)
def _sc_scatter(g_hbm, src_hbm, dst3_hbm, part_hbm, isrc_v, idst_v, buf0_v,
                buf1_v, semg0, semg1, acc_sh):
    c = lax.axis_index("c")
    s = lax.axis_index("s")
    wid = c * NS + s
    rbase = s * RPT

    pltpu.sync_copy(src_hbm.at[pl.ds(wid * EPW, EPW)], isrc_v)
    pltpu.sync_copy(dst3_hbm.at[wid], idst_v)

    @pl.when(c == 0)
    def _():
        pltpu.async_copy(g_hbm.at[isrc_v.at[pl.ds(0, BCH)]], buf0_v, semg0)
        pltpu.async_copy(g_hbm.at[isrc_v.at[pl.ds(BCH, BCH)]], buf1_v, semg1)
        pltpu.sync_copy(g_hbm.at[pl.ds(rbase, RPT)],
                        acc_sh.at[pl.ds(rbase, RPT)])

    @pl.when(c != 0)
    def _():
        zero16 = jnp.zeros((L,), jnp.float32)

        def _z(i, carry):
            buf0_v[lax.div(i, jnp.int32(D // L)),
                   pl.ds(lax.rem(i, jnp.int32(D // L)) * L, L)] = zero16
            return carry

        lax.fori_loop(0, BCH * D // L, _z, 0)

        def _zcp(k, carry):
            pltpu.sync_copy(buf0_v, acc_sh.at[pl.ds(rbase + k * BCH, BCH)])
            return carry

        lax.fori_loop(0, RPT // BCH, _zcp, 0)
        pltpu.async_copy(g_hbm.at[isrc_v.at[pl.ds(0, BCH)]], buf0_v, semg0)
        pltpu.async_copy(g_hbm.at[isrc_v.at[pl.ds(BCH, BCH)]], buf1_v, semg1)

    plsc.subcore_barrier()

    def _gather(j, buf, semg):
        pltpu.async_copy(g_hbm.at[isrc_v.at[pl.ds(j * BCH, BCH)]], buf, semg)

    def _gwait(j, buf, semg):
        pltpu.make_async_copy(g_hbm.at[isrc_v.at[pl.ds(j * BCH, BCH)]],
                              buf, semg).wait()

    def _edge(t, carry):
        a = 2 * t
        _gwait(a, buf0_v, semg0)
        pltpu.sync_copy(buf0_v, acc_sh.at[idst_v.at[a]], add=True)

        @pl.when(a + 2 < KCH)
        def _():
            _gather(a + 2, buf0_v, semg0)

        _gwait(a + 1, buf1_v, semg1)
        pltpu.sync_copy(buf1_v, acc_sh.at[idst_v.at[a + 1]], add=True)

        @pl.when(a + 3 < KCH)
        def _():
            _gather(a + 3, buf1_v, semg1)

        return carry

    lax.fori_loop(0, KCH // 2, _edge, 0)

    _gwait(KCH - 1, buf0_v, semg0)
    pltpu.sync_copy(buf0_v, acc_sh.at[idst_v.at[KCH - 1]], add=True)

    plsc.subcore_barrier()
    pltpu.sync_copy(acc_sh.at[pl.ds(rbase, RPT)],
                    part_hbm.at[c, pl.ds(rbase, RPT)])


def _tc_finish_body(part_ref, deg_ref, b_ref, out_ref):
    deg = jnp.sum(deg_ref[...], axis=0) + 1.0
    inv = lax.rsqrt(jnp.maximum(deg, 1e-9))
    p = part_ref[...]
    out_ref[...] = (p[0] + p[1]) * inv[:, None] + b_ref[...]


def _tc_finish(part, deg_parts, b2):
    blk = 2048
    grid = NPAD // blk
    return pl.pallas_call(
        _tc_finish_body,
        grid=(grid,),
        in_specs=[
            pl.BlockSpec((NC, blk, D), lambda i: (0, i, 0)),
            pl.BlockSpec((NW, blk), lambda i: (0, i)),
            pl.BlockSpec((1, D), lambda i: (0, 0)),
        ],
        out_specs=pl.BlockSpec((blk, D), lambda i: (i, 0)),
        out_shape=jax.ShapeDtypeStruct((N, D), jnp.float32),
    )(part, deg_parts, b2)


def kernel(x, edge_index, W, b):
    x2 = x.reshape(N, D)
    src_flat = edge_index[0]
    dst_flat = edge_index[1]
    dst3 = dst_flat.reshape(NW, KCH, BCH)
    b2 = b.reshape(1, D)

    deg_parts = _sc_degree(edge_index)
    g = _tc_transform(x2, W, deg_parts)
    part = _sc_scatter(g, src_flat, dst3)
    outp = _tc_finish(part, deg_parts, b2)
    return outp.reshape(1, 1, N, D)
